# baseline (speedup 1.0000x reference)
import functools

import jax
import jax.numpy as jnp
from jax import lax
from jax.experimental import pallas as pl
from jax.experimental.pallas import tpu as pltpu
from jax.experimental.pallas import tpu_sc as plsc

HIDDEN = 64
BATCH = 4096
HIST = 200
B_TOTAL = BATCH * HIST
NC, NS = 2, 16
NW = NC * NS
G = 128
CB = 256
GPC = CB // G
TPH = BATCH // CB
NTILE = HIST * TPH
TPW = NTILE // NW
L = 16


def _emb_body(x_hbm, table_hbm, out_hbm, idx_v, rows_v, outt_v, gat_sem, st_sem):
    wid = lax.axis_index("s") * NC + lax.axis_index("c")
    t0 = wid * TPW

    def tile_body(c, carry):
        t = t0 + c
        h = t // TPH
        b0 = (t % TPH) * CB

        pltpu.sync_copy(x_hbm.at[pl.ds(t * CB, CB)], idx_v)
        copies = []
        for j in range(GPC):
            copies.append(
                pltpu.async_copy(
                    table_hbm.at[idx_v.at[pl.ds(j * G, G)]],
                    rows_v.at[pl.ds(j * G, G)],
                    gat_sem,
                )
            )
        for cp in copies:
            cp.wait()

        def tr_body(c0, carry2):
            row_ids = c0 * L + lax.iota(jnp.int32, L)
            for f in range(HIDDEN):
                col_ids = jnp.full((L,), f, jnp.int32)
                vals = plsc.load_gather(rows_v, [row_ids, col_ids])
                outt_v[f, pl.ds(c0 * L, L)] = vals
            return carry2

        lax.fori_loop(0, CB // L, tr_body, 0)

        pltpu.sync_copy(outt_v, out_hbm.at[h, :, pl.ds(b0, CB)])
        return carry

    lax.fori_loop(0, TPW, tile_body, 0)


@jax.jit
def _embed(x_flat, table):
    mesh = plsc.VectorSubcoreMesh(core_axis_name="c", subcore_axis_name="s")
    k = pl.kernel(
        _emb_body,
        out_type=jax.ShapeDtypeStruct((HIST, HIDDEN, BATCH), jnp.float32),
        mesh=mesh,
        compiler_params=pltpu.CompilerParams(use_tc_tiling_on_sc=False,
                                             needs_layout_passes=False),
        scratch_types=[
            pltpu.VMEM((CB,), jnp.int32),
            pltpu.VMEM((CB, HIDDEN), jnp.float32),
            pltpu.VMEM((HIDDEN, CB), jnp.float32),
            pltpu.SemaphoreType.DMA,
            pltpu.SemaphoreType.DMA,
        ],
    )
    return k(x_flat, table)


def kernel(x, table):
    b, h = x.shape
    x_flat = x.T.reshape(B_TOTAL)
    out3 = _embed(x_flat, table)
    return out3.transpose(2, 0, 1)


# speedup vs baseline: 1.2299x; 1.2299x over previous
import functools

import jax
import jax.numpy as jnp
from jax import lax
from jax.experimental import pallas as pl
from jax.experimental.pallas import tpu as pltpu
from jax.experimental.pallas import tpu_sc as plsc

HIDDEN = 64
ROW2 = 2 * HIDDEN
BATCH = 4096
HIST = 200
VOCAB2 = 500000
B_TOTAL = BATCH * HIST
NC, NS = 2, 16
NW = NC * NS
G = 128
CB = 256
GPC = CB // G
TPH = BATCH // CB
NTILE = HIST * TPH
TPW = NTILE // NW
L = 16


def _emb_body(x_hbm, table_hbm, out_hbm,
              idx_v, idx2_v, rows_v, outt_v, gat_sem):
    wid = lax.axis_index("s") * NC + lax.axis_index("c")
    t0 = wid * TPW
    lane_iota = lax.iota(jnp.int32, L)

    def tile_body(c, carry):
        t = t0 + c
        h = t // TPH
        b0 = (t % TPH) * CB

        pltpu.sync_copy(x_hbm.at[pl.ds(t * CB, CB)], idx_v)

        @plsc.parallel_loop(0, CB // L)
        def _half(i):
            idx2_v[pl.ds(i * L, L)] = (
                lax.shift_right_logical(idx_v[pl.ds(i * L, L)], 1))

        copies = []
        for j in range(GPC):
            copies.append(
                pltpu.async_copy(
                    table_hbm.at[idx2_v.at[pl.ds(j * G, G)]],
                    rows_v.at[pl.ds(j * G, G)],
                    gat_sem,
                )
            )
        for cp in copies:
            cp.wait()

        @plsc.parallel_loop(0, CB // L)
        def _tr(c0):
            par = jnp.bitwise_and(idx_v[pl.ds(c0 * L, L)], 1)
            rows16 = c0 * L + lane_iota
            col_base = par * HIDDEN
            for f in range(HIDDEN):
                vals = plsc.load_gather(rows_v, [rows16, col_base + f])
                outt_v[f, pl.ds(c0 * L, L)] = vals

        pltpu.sync_copy(outt_v, out_hbm.at[h, :, pl.ds(b0, CB)])
        return carry

    lax.fori_loop(0, TPW, tile_body, 0)


@jax.jit
def _embed(x_flat, table2):
    mesh = plsc.VectorSubcoreMesh(core_axis_name="c", subcore_axis_name="s")
    k = pl.kernel(
        _emb_body,
        out_type=jax.ShapeDtypeStruct((HIST, HIDDEN, BATCH), jnp.float32),
        mesh=mesh,
        compiler_params=pltpu.CompilerParams(use_tc_tiling_on_sc=False,
                                             needs_layout_passes=False),
        scratch_types=[
            pltpu.VMEM((CB,), jnp.int32),
            pltpu.VMEM((CB,), jnp.int32),
            pltpu.VMEM((CB, ROW2), jnp.float32),
            pltpu.VMEM((HIDDEN, CB), jnp.float32),
            pltpu.SemaphoreType.DMA,
        ],
    )
    return k(x_flat, table2)


def kernel(x, table):
    b, h = x.shape
    x_flat = x.T.reshape(B_TOTAL)
    table2 = table.reshape(VOCAB2, ROW2)
    out3 = _embed(x_flat, table2)
    return out3.transpose(2, 0, 1)


# speedup vs baseline: 1.3323x; 1.0832x over previous
import functools

import jax
import jax.numpy as jnp
from jax import lax
from jax.experimental import pallas as pl
from jax.experimental.pallas import tpu as pltpu
from jax.experimental.pallas import tpu_sc as plsc

HIDDEN = 64
ROW2 = 2 * HIDDEN
BATCH = 4096
HIST = 200
VOCAB2 = 500000
B_TOTAL = BATCH * HIST
NC, NS = 2, 16
NW = NC * NS
G = 128
CB = 256
GPC = CB // G
TPH = BATCH // CB
NTILE = HIST * TPH
TPW = NTILE // NW
NPAIR = TPW // 2
L = 16


def _emb_body(x_hbm, table_hbm, out_hbm,
              idx_v, idx2_v, rows_v, outt_v, g0, g1, s0, s1):
    wid = lax.axis_index("s") * NC + lax.axis_index("c")
    t0 = wid * TPW
    lane_iota = lax.iota(jnp.int32, L)
    gsem = (g0, g1)
    ssem = (s0, s1)

    def stage_idx(c, b):
        pltpu.sync_copy(x_hbm.at[pl.ds((t0 + c) * CB, CB)], idx_v.at[b])

        @plsc.parallel_loop(0, CB // L)
        def _half(i):
            idx2_v[b, pl.ds(i * L, L)] = (
                lax.shift_right_logical(idx_v[b, pl.ds(i * L, L)], 1))

    def fire_gathers(b):
        for j in range(GPC):
            pltpu.async_copy(
                table_hbm.at[idx2_v.at[b, pl.ds(j * G, G)]],
                rows_v.at[b, pl.ds(j * G, G)],
                gsem[b],
            )

    def wait_gathers(b):
        for j in range(GPC):
            pltpu.make_async_copy(
                table_hbm.at[idx2_v.at[b, pl.ds(j * G, G)]],
                rows_v.at[b, pl.ds(j * G, G)],
                gsem[b],
            ).wait()

    def store_descr(c, b):
        t = t0 + c
        h = t // TPH
        b0 = (t % TPH) * CB
        return pltpu.make_async_copy(
            outt_v.at[b], out_hbm.at[h, :, pl.ds(b0, CB)], ssem[b])

    def transpose(b):
        @plsc.parallel_loop(0, CB // L, unroll=4)
        def _tr(c0):
            par = jnp.bitwise_and(idx_v[b, pl.ds(c0 * L, L)], 1)
            rows16 = c0 * L + lane_iota
            col_base = par * HIDDEN
            for f in range(HIDDEN):
                vals = plsc.load_gather(rows_v.at[b], [rows16, col_base + f])
                outt_v[b, f, pl.ds(c0 * L, L)] = vals

    for b in range(2):
        stage_idx(b, b)
        fire_gathers(b)

    def pair_body(p, carry):
        for b in range(2):
            c = 2 * p + b
            wait_gathers(b)

            @pl.when(p > 0)
            def _():
                store_descr(c - 2, b).wait()

            transpose(b)
            store_descr(c, b).start()

            @pl.when(p < NPAIR - 1)
            def _():
                stage_idx(c + 2, b)
                fire_gathers(b)
        return carry

    lax.fori_loop(0, NPAIR, pair_body, 0)

    for b in range(2):
        store_descr(TPW - 2 + b, b).wait()


@jax.jit
def _embed(x_flat, table2):
    mesh = plsc.VectorSubcoreMesh(core_axis_name="c", subcore_axis_name="s")
    k = pl.kernel(
        _emb_body,
        out_type=jax.ShapeDtypeStruct((HIST, HIDDEN, BATCH), jnp.float32),
        mesh=mesh,
        compiler_params=pltpu.CompilerParams(use_tc_tiling_on_sc=False,
                                             needs_layout_passes=False),
        scratch_types=[
            pltpu.VMEM((2, CB), jnp.int32),
            pltpu.VMEM((2, CB), jnp.int32),
            pltpu.VMEM((2, CB, ROW2), jnp.float32),
            pltpu.VMEM((2, HIDDEN, CB), jnp.float32),
            pltpu.SemaphoreType.DMA,
            pltpu.SemaphoreType.DMA,
            pltpu.SemaphoreType.DMA,
            pltpu.SemaphoreType.DMA,
        ],
    )
    return k(x_flat, table2)


def kernel(x, table):
    b, h = x.shape
    x_flat = x.T.reshape(B_TOTAL)
    table2 = table.reshape(VOCAB2, ROW2)
    out3 = _embed(x_flat, table2)
    return out3.transpose(2, 0, 1)


# speedup vs baseline: 1.3634x; 1.0233x over previous
import functools

import jax
import jax.numpy as jnp
from jax import lax
from jax.experimental import pallas as pl
from jax.experimental.pallas import tpu as pltpu
from jax.experimental.pallas import tpu_sc as plsc

HIDDEN = 64
ROW2 = 2 * HIDDEN
BATCH = 4096
HIST = 200
VOCAB2 = 500000
B_TOTAL = BATCH * HIST
NC, NS = 2, 16
NW = NC * NS
G = 128
CB = 256
GPC = CB // G
TPH = BATCH // CB
NTILE = HIST * TPH
TPW = NTILE // NW
NPAIR = TPW // 2
L = 16


def _emb_body(x_hbm, table_hbm, out_hbm,
              idx_v0, idx_v1, idx2_v0, idx2_v1, rows_v0, rows_v1,
              outt_v0, outt_v1, g0, g1, s0, s1):
    wid = lax.axis_index("s") * NC + lax.axis_index("c")
    t0 = wid * TPW
    lane_iota = lax.iota(jnp.int32, L)
    idx_v = (idx_v0, idx_v1)
    idx2_v = (idx2_v0, idx2_v1)
    rows_v = (rows_v0, rows_v1)
    outt_v = (outt_v0, outt_v1)
    gsem = (g0, g1)
    ssem = (s0, s1)

    def stage_idx(c, b):
        pltpu.sync_copy(x_hbm.at[pl.ds((t0 + c) * CB, CB)], idx_v[b])

        @plsc.parallel_loop(0, CB // L)
        def _half(i):
            idx2_v[b][pl.ds(i * L, L)] = (
                lax.shift_right_logical(idx_v[b][pl.ds(i * L, L)], 1))

    def fire_gathers(b):
        for j in range(GPC):
            pltpu.async_copy(
                table_hbm.at[idx2_v[b].at[pl.ds(j * G, G)]],
                rows_v[b].at[pl.ds(j * G, G)],
                gsem[b],
            )

    def wait_gathers(b):
        for j in range(GPC):
            pltpu.make_async_copy(
                table_hbm.at[idx2_v[b].at[pl.ds(j * G, G)]],
                rows_v[b].at[pl.ds(j * G, G)],
                gsem[b],
            ).wait()

    def store_descr(c, b):
        t = t0 + c
        h = t // TPH
        b0 = (t % TPH) * CB
        return pltpu.make_async_copy(
            outt_v[b], out_hbm.at[h, :, pl.ds(b0, CB)], ssem[b])

    def transpose(b):
        @plsc.parallel_loop(0, CB // L, unroll=2)
        def _tr(c0):
            par = jnp.bitwise_and(idx_v[b][pl.ds(c0 * L, L)], 1)
            rows16 = c0 * L + lane_iota
            col_base = par * HIDDEN
            for f in range(HIDDEN):
                vals = plsc.load_gather(rows_v[b], [rows16, col_base + f])
                outt_v[b][f, pl.ds(c0 * L, L)] = vals

    for b in range(2):
        stage_idx(b, b)
        fire_gathers(b)

    def pair_body(p, carry):
        for b in range(2):
            c = 2 * p + b
            wait_gathers(b)

            @pl.when(p > 0)
            def _():
                store_descr(c - 2, b).wait()

            transpose(b)
            store_descr(c, b).start()

            @pl.when(p < NPAIR - 1)
            def _():
                stage_idx(c + 2, b)
                fire_gathers(b)
        return carry

    lax.fori_loop(0, NPAIR, pair_body, 0)

    for b in range(2):
        store_descr(TPW - 2 + b, b).wait()


@jax.jit
def _embed(x_flat, table2):
    mesh = plsc.VectorSubcoreMesh(core_axis_name="c", subcore_axis_name="s")
    k = pl.kernel(
        _emb_body,
        out_type=jax.ShapeDtypeStruct((HIST, HIDDEN, BATCH), jnp.float32),
        mesh=mesh,
        compiler_params=pltpu.CompilerParams(use_tc_tiling_on_sc=False,
                                             needs_layout_passes=False),
        scratch_types=[
            pltpu.VMEM((CB,), jnp.int32),
            pltpu.VMEM((CB,), jnp.int32),
            pltpu.VMEM((CB,), jnp.int32),
            pltpu.VMEM((CB,), jnp.int32),
            pltpu.VMEM((CB, ROW2), jnp.float32),
            pltpu.VMEM((CB, ROW2), jnp.float32),
            pltpu.VMEM((HIDDEN, CB), jnp.float32),
            pltpu.VMEM((HIDDEN, CB), jnp.float32),
            pltpu.SemaphoreType.DMA,
            pltpu.SemaphoreType.DMA,
            pltpu.SemaphoreType.DMA,
            pltpu.SemaphoreType.DMA,
        ],
    )
    return k(x_flat, table2)


def kernel(x, table):
    b, h = x.shape
    x_flat = x.T.reshape(B_TOTAL)
    table2 = table.reshape(VOCAB2, ROW2)
    out3 = _embed(x_flat, table2)
    return out3.transpose(2, 0, 1)


# speedup vs baseline: 1.4568x; 1.0685x over previous
import functools

import jax
import jax.numpy as jnp
from jax import lax
from jax.experimental import pallas as pl
from jax.experimental.pallas import tpu as pltpu
from jax.experimental.pallas import tpu_sc as plsc

HIDDEN = 64
ROW2 = 2 * HIDDEN
BATCH = 4096
HIST = 200
VOCAB2 = 500000
B_TOTAL = BATCH * HIST
NC, NS = 2, 16
NW = NC * NS
G = 128
CB = 256
GPC = CB // G
TPH = BATCH // CB
NTILE = HIST * TPH
TPW = NTILE // NW
NPAIR = TPW // 2
L = 16


def _emb_body(x_hbm, table_hbm, out_hbm,
              idx_v0, idx_v1, rows_v0, rows_v1,
              outt_v0, outt_v1, g0, g1, s0, s1):
    wid = lax.axis_index("s") * NC + lax.axis_index("c")
    t0 = wid * TPW
    lane_iota = lax.iota(jnp.int32, L)
    idx_v = (idx_v0, idx_v1)
    rows_v = (rows_v0, rows_v1)
    outt_v = (outt_v0, outt_v1)
    gsem = (g0, g1)
    ssem = (s0, s1)

    def stage_idx(c, b):
        pltpu.sync_copy(x_hbm.at[pl.ds((t0 + c) * CB, CB)], idx_v[b])

    def fire_gathers(b):
        for j in range(GPC):
            pltpu.async_copy(
                table_hbm.at[idx_v[b].at[pl.ds(j * G, G)]],
                rows_v[b].at[pl.ds(j * G, G)],
                gsem[b],
            )

    def wait_gathers(b):
        for j in range(GPC):
            pltpu.make_async_copy(
                table_hbm.at[idx_v[b].at[pl.ds(j * G, G)]],
                rows_v[b].at[pl.ds(j * G, G)],
                gsem[b],
            ).wait()

    def store_descr(c, b):
        t = t0 + c
        h = t // TPH
        b0 = (t % TPH) * CB
        return pltpu.make_async_copy(
            outt_v[b], out_hbm.at[h, :, pl.ds(b0, CB)], ssem[b])

    fvecs = [k * L + lane_iota for k in range(HIDDEN // L)]

    def transpose(b):
        @plsc.parallel_loop(0, CB, unroll=4)
        def _tr(c):
            cvec = jnp.full((L,), c, jnp.int32)
            for k in range(HIDDEN // L):
                vals = rows_v[b][c, pl.ds(k * L, L)]
                plsc.store_scatter(outt_v[b], [fvecs[k], cvec], vals)

    for b in range(2):
        stage_idx(b, b)
        fire_gathers(b)

    def pair_body(p, carry):
        for b in range(2):
            c = 2 * p + b
            wait_gathers(b)

            @pl.when(p > 0)
            def _():
                store_descr(c - 2, b).wait()

            transpose(b)
            store_descr(c, b).start()

            @pl.when(p < NPAIR - 1)
            def _():
                stage_idx(c + 2, b)
                fire_gathers(b)
        return carry

    lax.fori_loop(0, NPAIR, pair_body, 0)

    for b in range(2):
        store_descr(TPW - 2 + b, b).wait()


@jax.jit
def _embed(x_flat, table):
    mesh = plsc.VectorSubcoreMesh(core_axis_name="c", subcore_axis_name="s")
    k = pl.kernel(
        _emb_body,
        out_type=jax.ShapeDtypeStruct((HIST, HIDDEN, BATCH), jnp.float32),
        mesh=mesh,
        compiler_params=pltpu.CompilerParams(use_tc_tiling_on_sc=False,
                                             needs_layout_passes=False),
        scratch_types=[
            pltpu.VMEM((CB,), jnp.int32),
            pltpu.VMEM((CB,), jnp.int32),
            pltpu.VMEM((CB, HIDDEN), jnp.float32),
            pltpu.VMEM((CB, HIDDEN), jnp.float32),
            pltpu.VMEM((HIDDEN, CB), jnp.float32),
            pltpu.VMEM((HIDDEN, CB), jnp.float32),
            pltpu.SemaphoreType.DMA,
            pltpu.SemaphoreType.DMA,
            pltpu.SemaphoreType.DMA,
            pltpu.SemaphoreType.DMA,
        ],
    )
    return k(x_flat, table)


def kernel(x, table):
    b, h = x.shape
    x_flat = x.T.reshape(B_TOTAL)
    out3 = _embed(x_flat, table)
    return out3.transpose(2, 0, 1)
